# baseline (speedup 1.0000x reference)
import functools

import jax
import jax.numpy as jnp
from jax import lax
from jax.experimental import pallas as pl
from jax.experimental.pallas import tpu as pltpu
from jax.experimental.pallas import tpu_sc as plsc

N = 10000
R = 10240
E = 320000
EP = 327680
NPAD = 64
BM = 640
GB = 64
RPT = R // 16

_MESH = dict(core_axis_name="c", subcore_axis_name="s")


def _build_degree():
    ept = EP // 32

    @functools.partial(
        pl.kernel,
        mesh=plsc.VectorSubcoreMesh(**_MESH),
        out_type=jax.ShapeDtypeStruct((32, R), jnp.float32),
        scratch_types=[
            pltpu.VMEM((R,), jnp.float32),
            pltpu.VMEM((ept,), jnp.int32),
        ],
        compiler_params=pltpu.CompilerParams(needs_layout_passes=False),
    )
    def deg_kernel(dst_hbm, out_hbm, hist, dstbuf):
        c = lax.axis_index("c")
        s = lax.axis_index("s")
        wid = c * 16 + s

        def zero(i, carry):
            hist[pl.ds(i * 16, 16)] = jnp.zeros((16,), jnp.float32)
            return carry
        lax.fori_loop(0, R // 16, zero, 0)

        pltpu.sync_copy(dst_hbm.at[pl.ds(wid * ept, ept)], dstbuf)

        ones = jnp.ones((16,), jnp.float32)

        def body(i, carry):
            idx = dstbuf[pl.ds(i * 16, 16)]
            plsc.addupdate_scatter(hist, [idx], ones)
            return carry
        lax.fori_loop(0, ept // 16, body, 0)

        pltpu.sync_copy(hist, out_hbm.at[wid])

    return deg_kernel


def _build_scatter(split_edges):
    ept = EP // 32 if split_edges else EP // 16
    nb = ept // GB
    CH = 40
    nch = nb // CH
    NBUF = 4
    gc = CH // NBUF

    @functools.partial(
        pl.kernel,
        mesh=plsc.VectorSubcoreMesh(**_MESH),
        out_type=[
            jax.ShapeDtypeStruct((R, 128), jnp.float32),
            jax.ShapeDtypeStruct((R, 128), jnp.float32),
        ],
        scratch_types=[
            pltpu.VMEM((CH, GB), jnp.int32),
            pltpu.VMEM((CH, GB), jnp.int32),
            [pltpu.VMEM((GB, 128), jnp.float32)] * NBUF,
            [pltpu.SemaphoreType.DMA] * NBUF,
            pltpu.VMEM_SHARED((R, 128), jnp.float32),
        ],
    )
    def scat_kernel(ta, tb, src_hbm, dst_hbm, oa, ob,
                    srcb, dstb, bufs, sems, acc):
        c = lax.axis_index("c")
        s = lax.axis_index("s")

        def zr(i, carry):
            bufs[0][i // 8, pl.ds((i % 8) * 16, 16)] = jnp.zeros((16,), jnp.float32)
            return carry
        lax.fori_loop(0, GB * 8, zr, 0)
        for kk in range(RPT // GB):
            pltpu.sync_copy(bufs[0], acc.at[pl.ds(s * RPT + kk * GB, GB)])
        plsc.subcore_barrier()

        if split_edges:
            row_base = (c * 16 + s) * nb
        else:
            row_base = s * nb

        def gstart(j, b):
            @pl.when(c == 0)
            def _():
                pltpu.async_copy(ta.at[srcb.at[b]], bufs[j], sems[j])

            @pl.when(c == 1)
            def _():
                pltpu.async_copy(tb.at[srcb.at[b]], bufs[j], sems[j])

        def gwait(j):
            pltpu.make_async_copy(ta.at[srcb.at[0]], bufs[j], sems[j]).wait()

        def chunk(ch, carry):
            row0 = row_base + ch * CH
            pltpu.sync_copy(src_hbm.at[pl.ds(row0, CH)], srcb)
            pltpu.sync_copy(dst_hbm.at[pl.ds(row0, CH)], dstb)
            gstart(0, 0)
            gstart(1, 1)

            def group(bg, carry2):
                b0 = bg * NBUF
                for j in range(NBUF):
                    b = b0 + j
                    gwait(j)

                    @pl.when(b + 2 < CH)
                    def _():
                        gstart((j + 2) % NBUF, b + 2)

                    pltpu.sync_copy(bufs[j], acc.at[dstb.at[b]], add=True)
                return carry2
            lax.fori_loop(0, gc, group, 0)
            return carry
        lax.fori_loop(0, nch, chunk, 0)

        plsc.subcore_barrier()
        ob_base = s * RPT

        @pl.when(c == 0)
        def _():
            pltpu.sync_copy(acc.at[pl.ds(ob_base, RPT)], oa.at[pl.ds(ob_base, RPT)])

        @pl.when(c == 1)
        def _():
            pltpu.sync_copy(acc.at[pl.ds(ob_base, RPT)], ob.at[pl.ds(ob_base, RPT)])

    return scat_kernel


_deg_call = _build_degree()
_scatter_halves = _build_scatter(split_edges=False)
_scatter_split = _build_scatter(split_edges=True)


def _dinv_from_cnt(c_ref):
    deg = 1.0 + jnp.sum(c_ref[...], axis=0)
    return lax.rsqrt(deg)


def _tc1(xp, W1, cnt):
    def body(x_ref, w_ref, c_ref, ga_ref, gb_ref):
        dinv = _dinv_from_cnt(c_ref)
        h = jnp.dot(x_ref[...], w_ref[...], preferred_element_type=jnp.float32)
        g = h * dinv[:, None]
        ga_ref[...] = g[:, :128]
        gb_ref[...] = g[:, 128:]

    return pl.pallas_call(
        body,
        grid=(R // BM,),
        in_specs=[
            pl.BlockSpec((BM, 128), lambda i: (i, 0)),
            pl.BlockSpec((128, 256), lambda i: (0, 0)),
            pl.BlockSpec((32, BM), lambda i: (0, i)),
        ],
        out_specs=[
            pl.BlockSpec((BM, 128), lambda i: (i, 0)),
            pl.BlockSpec((BM, 128), lambda i: (i, 0)),
        ],
        out_shape=[jax.ShapeDtypeStruct((R, 128), jnp.float32)] * 2,
    )(xp, W1, cnt)


def _tc2(s1a, s1b, g1a, g1b, cnt, W2, b1):
    def body(sa, sb, ga, gb, c_ref, w_ref, b_ref, out_ref):
        dinv = _dinv_from_cnt(c_ref)
        ha = (sa[...] + ga[...]) * dinv[:, None]
        hb = (sb[...] + gb[...]) * dinv[:, None]
        z = jnp.concatenate([ha, hb], axis=1) + b_ref[...]
        z = jnp.maximum(z, 0.0)
        h2 = jnp.dot(z, w_ref[...], preferred_element_type=jnp.float32)
        out_ref[...] = h2 * dinv[:, None]

    blk = pl.BlockSpec((BM, 128), lambda i: (i, 0))
    return pl.pallas_call(
        body,
        grid=(R // BM,),
        in_specs=[
            blk, blk, blk, blk,
            pl.BlockSpec((32, BM), lambda i: (0, i)),
            pl.BlockSpec((256, 128), lambda i: (0, 0)),
            pl.BlockSpec((1, 256), lambda i: (0, 0)),
        ],
        out_specs=blk,
        out_shape=jax.ShapeDtypeStruct((R, 128), jnp.float32),
    )(s1a, s1b, g1a, g1b, cnt, W2, b1)


def _tc3(s2a, s2b, g2, cnt, Wf, b2, bf):
    def body(sa, sb, g_ref, c_ref, w_ref, b2_ref, bf_ref, out_ref):
        dinv = _dinv_from_cnt(c_ref)
        conv2 = (sa[...] + sb[...] + g_ref[...]) * dinv[:, None] + b2_ref[...]
        out_ref[...] = (
            jnp.dot(conv2, w_ref[...], preferred_element_type=jnp.float32)
            + bf_ref[...]
        )

    blk = pl.BlockSpec((BM, 128), lambda i: (i, 0))
    return pl.pallas_call(
        body,
        grid=(R // BM,),
        in_specs=[
            blk, blk, blk,
            pl.BlockSpec((32, BM), lambda i: (0, i)),
            pl.BlockSpec((128, 128), lambda i: (0, 0)),
            pl.BlockSpec((1, 128), lambda i: (0, 0)),
            pl.BlockSpec((1, 128), lambda i: (0, 0)),
        ],
        out_specs=blk,
        out_shape=jax.ShapeDtypeStruct((R, 128), jnp.float32),
    )(s2a, s2b, g2, cnt, Wf, b2, bf)


def kernel(x, edge_index, W1, b1, W2, b2, Wf, bf):
    src = edge_index[0].astype(jnp.int32)
    dst = edge_index[1].astype(jnp.int32)
    pad_idx = (jnp.arange(EP - E, dtype=jnp.int32) % NPAD) + N
    srcp = jnp.concatenate([src, pad_idx]).reshape(EP // GB, GB)
    dstp_flat = jnp.concatenate([dst, pad_idx])
    dstp = dstp_flat.reshape(EP // GB, GB)
    xp = jnp.pad(x, ((0, R - N), (0, 0)))

    cnt = _deg_call(dstp_flat)
    g1a, g1b = _tc1(xp, W1, cnt)
    s1a, s1b = _scatter_halves(g1a, g1b, srcp, dstp)
    g2 = _tc2(s1a, s1b, g1a, g1b, cnt, W2, b1.reshape(1, 256))
    s2a, s2b = _scatter_split(g2, g2, srcp, dstp)
    out = _tc3(s2a, s2b, g2, cnt, Wf, b2.reshape(1, 128), bf.reshape(1, 128))
    return out[:N]


# speedup vs baseline: 1.0332x; 1.0332x over previous
import functools

import jax
import jax.numpy as jnp
from jax import lax
from jax.experimental import pallas as pl
from jax.experimental.pallas import tpu as pltpu
from jax.experimental.pallas import tpu_sc as plsc

N = 10000
R = 10240
E = 320000
EP = 327680
NPAD = 64
BM = 640
GB = 128
RPT = R // 16

_MESH = dict(core_axis_name="c", subcore_axis_name="s")


def _build_degree():
    ept = EP // 32

    @functools.partial(
        pl.kernel,
        mesh=plsc.VectorSubcoreMesh(**_MESH),
        out_type=jax.ShapeDtypeStruct((32, R), jnp.float32),
        scratch_types=[
            pltpu.VMEM((R,), jnp.float32),
            pltpu.VMEM((ept,), jnp.int32),
        ],
        compiler_params=pltpu.CompilerParams(needs_layout_passes=False),
    )
    def deg_kernel(dst_hbm, out_hbm, hist, dstbuf):
        c = lax.axis_index("c")
        s = lax.axis_index("s")
        wid = c * 16 + s

        def zero(i, carry):
            hist[pl.ds(i * 16, 16)] = jnp.zeros((16,), jnp.float32)
            return carry
        lax.fori_loop(0, R // 16, zero, 0)

        pltpu.sync_copy(dst_hbm.at[pl.ds(wid * ept, ept)], dstbuf)

        ones = jnp.ones((16,), jnp.float32)

        def body(i, carry):
            idx = dstbuf[pl.ds(i * 16, 16)]
            plsc.addupdate_scatter(hist, [idx], ones)
            return carry
        lax.fori_loop(0, ept // 16, body, 0)

        pltpu.sync_copy(hist, out_hbm.at[wid])

    return deg_kernel


def _build_scatter(split_edges):
    ept = EP // 32 if split_edges else EP // 16
    nb = ept // GB
    CH = 40
    nch = nb // CH
    NBUF = 2
    gc = CH // NBUF

    @functools.partial(
        pl.kernel,
        mesh=plsc.VectorSubcoreMesh(**_MESH),
        out_type=[
            jax.ShapeDtypeStruct((R, 128), jnp.float32),
            jax.ShapeDtypeStruct((R, 128), jnp.float32),
        ],
        scratch_types=[
            pltpu.VMEM((CH, GB), jnp.int32),
            pltpu.VMEM((CH, GB), jnp.int32),
            [pltpu.VMEM((GB, 128), jnp.float32)] * NBUF,
            [pltpu.SemaphoreType.DMA] * NBUF,
            pltpu.VMEM_SHARED((R, 128), jnp.float32),
        ],
    )
    def scat_kernel(ta, tb, src_hbm, dst_hbm, oa, ob,
                    srcb, dstb, bufs, sems, acc):
        c = lax.axis_index("c")
        s = lax.axis_index("s")

        def zr(i, carry):
            bufs[0][i // 8, pl.ds((i % 8) * 16, 16)] = jnp.zeros((16,), jnp.float32)
            return carry
        lax.fori_loop(0, GB * 8, zr, 0)
        for kk in range(RPT // GB):
            pltpu.sync_copy(bufs[0], acc.at[pl.ds(s * RPT + kk * GB, GB)])
        plsc.subcore_barrier()

        if split_edges:
            row_base = (c * 16 + s) * nb
        else:
            row_base = s * nb

        def gstart(j, b):
            @pl.when(c == 0)
            def _():
                pltpu.async_copy(ta.at[srcb.at[b]], bufs[j], sems[j])

            @pl.when(c == 1)
            def _():
                pltpu.async_copy(tb.at[srcb.at[b]], bufs[j], sems[j])

        def gwait(j):
            pltpu.make_async_copy(ta.at[srcb.at[0]], bufs[j], sems[j]).wait()

        def chunk(ch, carry):
            row0 = row_base + ch * CH
            pltpu.sync_copy(src_hbm.at[pl.ds(row0, CH)], srcb)
            pltpu.sync_copy(dst_hbm.at[pl.ds(row0, CH)], dstb)
            gstart(0, 0)

            def group(bg, carry2):
                b0 = bg * 2
                gstart(1, b0 + 1)
                gwait(0)
                pltpu.sync_copy(bufs[0], acc.at[dstb.at[b0]], add=True)

                @pl.when(b0 + 2 < CH)
                def _():
                    gstart(0, b0 + 2)

                gwait(1)
                pltpu.sync_copy(bufs[1], acc.at[dstb.at[b0 + 1]], add=True)
                return carry2
            lax.fori_loop(0, gc, group, 0)
            return carry
        lax.fori_loop(0, nch, chunk, 0)

        plsc.subcore_barrier()
        ob_base = s * RPT

        @pl.when(c == 0)
        def _():
            pltpu.sync_copy(acc.at[pl.ds(ob_base, RPT)], oa.at[pl.ds(ob_base, RPT)])

        @pl.when(c == 1)
        def _():
            pltpu.sync_copy(acc.at[pl.ds(ob_base, RPT)], ob.at[pl.ds(ob_base, RPT)])

    return scat_kernel


_deg_call = _build_degree()
_scatter_halves = _build_scatter(split_edges=False)
_scatter_split = _build_scatter(split_edges=True)


def _dinv_from_cnt(c_ref):
    deg = 1.0 + jnp.sum(c_ref[...], axis=0)
    return lax.rsqrt(deg)


def _tc1(xp, W1, cnt):
    def body(x_ref, w_ref, c_ref, ga_ref, gb_ref):
        dinv = _dinv_from_cnt(c_ref)
        h = jnp.dot(x_ref[...], w_ref[...], preferred_element_type=jnp.float32)
        g = h * dinv[:, None]
        ga_ref[...] = g[:, :128]
        gb_ref[...] = g[:, 128:]

    return pl.pallas_call(
        body,
        grid=(R // BM,),
        in_specs=[
            pl.BlockSpec((BM, 128), lambda i: (i, 0)),
            pl.BlockSpec((128, 256), lambda i: (0, 0)),
            pl.BlockSpec((32, BM), lambda i: (0, i)),
        ],
        out_specs=[
            pl.BlockSpec((BM, 128), lambda i: (i, 0)),
            pl.BlockSpec((BM, 128), lambda i: (i, 0)),
        ],
        out_shape=[jax.ShapeDtypeStruct((R, 128), jnp.float32)] * 2,
    )(xp, W1, cnt)


def _tc2(s1a, s1b, g1a, g1b, cnt, W2, b1):
    def body(sa, sb, ga, gb, c_ref, w_ref, b_ref, out_ref):
        dinv = _dinv_from_cnt(c_ref)
        ha = (sa[...] + ga[...]) * dinv[:, None]
        hb = (sb[...] + gb[...]) * dinv[:, None]
        z = jnp.concatenate([ha, hb], axis=1) + b_ref[...]
        z = jnp.maximum(z, 0.0)
        h2 = jnp.dot(z, w_ref[...], preferred_element_type=jnp.float32)
        out_ref[...] = h2 * dinv[:, None]

    blk = pl.BlockSpec((BM, 128), lambda i: (i, 0))
    return pl.pallas_call(
        body,
        grid=(R // BM,),
        in_specs=[
            blk, blk, blk, blk,
            pl.BlockSpec((32, BM), lambda i: (0, i)),
            pl.BlockSpec((256, 128), lambda i: (0, 0)),
            pl.BlockSpec((1, 256), lambda i: (0, 0)),
        ],
        out_specs=blk,
        out_shape=jax.ShapeDtypeStruct((R, 128), jnp.float32),
    )(s1a, s1b, g1a, g1b, cnt, W2, b1)


def _tc3(s2a, s2b, g2, cnt, Wf, b2, bf):
    def body(sa, sb, g_ref, c_ref, w_ref, b2_ref, bf_ref, out_ref):
        dinv = _dinv_from_cnt(c_ref)
        conv2 = (sa[...] + sb[...] + g_ref[...]) * dinv[:, None] + b2_ref[...]
        out_ref[...] = (
            jnp.dot(conv2, w_ref[...], preferred_element_type=jnp.float32)
            + bf_ref[...]
        )

    blk = pl.BlockSpec((BM, 128), lambda i: (i, 0))
    return pl.pallas_call(
        body,
        grid=(R // BM,),
        in_specs=[
            blk, blk, blk,
            pl.BlockSpec((32, BM), lambda i: (0, i)),
            pl.BlockSpec((128, 128), lambda i: (0, 0)),
            pl.BlockSpec((1, 128), lambda i: (0, 0)),
            pl.BlockSpec((1, 128), lambda i: (0, 0)),
        ],
        out_specs=blk,
        out_shape=jax.ShapeDtypeStruct((N, 128), jnp.float32),
    )(s2a, s2b, g2, cnt, Wf, b2, bf)


def kernel(x, edge_index, W1, b1, W2, b2, Wf, bf):
    src = edge_index[0].astype(jnp.int32)
    dst = edge_index[1].astype(jnp.int32)
    pad_idx = (jnp.arange(EP - E, dtype=jnp.int32) % NPAD) + N
    srcp = jnp.concatenate([src, pad_idx]).reshape(EP // GB, GB)
    dstp_flat = jnp.concatenate([dst, pad_idx])
    dstp = dstp_flat.reshape(EP // GB, GB)
    xp = jnp.pad(x, ((0, R - N), (0, 0)))

    cnt = _deg_call(dstp_flat)
    g1a, g1b = _tc1(xp, W1, cnt)
    s1a, s1b = _scatter_halves(g1a, g1b, srcp, dstp)
    g2 = _tc2(s1a, s1b, g1a, g1b, cnt, W2, b1.reshape(1, 256))
    s2a, s2b = _scatter_split(g2, g2, srcp, dstp)
    return _tc3(s2a, s2b, g2, cnt, Wf, b2.reshape(1, 128), bf.reshape(1, 128))
